# Initial kernel scaffold; baseline (speedup 1.0000x reference)
import functools

import jax
import jax.numpy as jnp
import numpy as np
from jax.experimental import pallas as pl
from jax.experimental.pallas import tpu as pltpu

_HID = 128
_G4 = 4 * _HID


def _lstm3_body(C, g_ref, wihT_ref, whhT_ref, y_ref, hn_ref, yscr, gxscr, hcscr):
    l = pl.program_id(0)
    c = pl.program_id(1)
    base = c * C

    xin = jnp.where(l == 0, g_ref[...], yscr[pl.ds(base, C), :])
    gxscr[...] = jnp.dot(xin, wihT_ref[0], preferred_element_type=jnp.float32)

    @pl.when(c == 0)
    def _():
        hcscr[...] = jnp.zeros_like(hcscr)

    whh = whhT_ref[0]

    def step(t, carry):
        h, cc = carry
        gates = gxscr[pl.ds(t, 1), :] + jnp.dot(h, whh,
                                                preferred_element_type=jnp.float32)
        i = jax.nn.sigmoid(gates[:, 0:_HID])
        f = jax.nn.sigmoid(gates[:, _HID:2 * _HID])
        gg = jnp.tanh(gates[:, 2 * _HID:3 * _HID])
        o = jax.nn.sigmoid(gates[:, 3 * _HID:4 * _HID])
        cc = f * cc + i * gg
        h = o * jnp.tanh(cc)
        yscr[pl.ds(base + t, 1), :] = h
        return (h, cc)

    h, cc = jax.lax.fori_loop(0, C, step, (hcscr[0:1, :], hcscr[1:2, :]))
    hcscr[0:1, :] = h
    hcscr[1:2, :] = cc
    y_ref[...] = yscr[pl.ds(base, C), :]

    @pl.when(c == pl.num_programs(1) - 1)
    def _():
        hn_ref[pl.ds(l, 1), :] = h


def _lstm3(g, wihT, whhT, C):
    n = g.shape[0]
    nc = n // C
    body = functools.partial(_lstm3_body, C)
    y2, hn = pl.pallas_call(
        body,
        grid=(3, nc),
        in_specs=[
            pl.BlockSpec((C, _HID), lambda l, c: (c, 0)),
            pl.BlockSpec((1, _HID, _G4), lambda l, c: (l, 0, 0)),
            pl.BlockSpec((1, _HID, _G4), lambda l, c: (l, 0, 0)),
        ],
        out_specs=[
            pl.BlockSpec((C, _HID), lambda l, c: (c, 0)),
            pl.BlockSpec((8, _HID), lambda l, c: (0, 0)),
        ],
        out_shape=[
            jax.ShapeDtypeStruct((n, _HID), jnp.float32),
            jax.ShapeDtypeStruct((8, _HID), jnp.float32),
        ],
        scratch_shapes=[
            pltpu.VMEM((n, _HID), jnp.float32),
            pltpu.VMEM((C, _G4), jnp.float32),
            pltpu.VMEM((8, _HID), jnp.float32),
        ],
    )(g, wihT, whhT)
    return y2, hn[0:3]


def _head_body(C, y2_ref, hn_ref, linmp_ref, linap_ref, linhn_ref, linb_ref,
               out_ref, accs):
    c = pl.program_id(0)

    @pl.when(c == 0)
    def _():
        accs[...] = jnp.zeros_like(accs)

    y2 = y2_ref[...]
    rolled = pltpu.roll(y2, -1, 1)
    sel = jax.lax.broadcasted_iota(jnp.int32, (C, _HID), 1) % 2 == 0
    smat = (jax.lax.broadcasted_iota(jnp.int32, (_HID, _HID // 2), 0) ==
            2 * jax.lax.broadcasted_iota(jnp.int32, (_HID, _HID // 2), 1)
            ).astype(jnp.float32)
    pairmax = jnp.maximum(y2, rolled)
    pairsum = y2 + rolled
    mp = jnp.dot(jnp.where(sel, pairmax, 0.0), smat,
                 preferred_element_type=jnp.float32)
    ap = 0.5 * jnp.dot(jnp.where(sel, pairsum, 0.0), smat,
                       preferred_element_type=jnp.float32)

    accs[0:3, 0:64] += jnp.sum(linmp_ref[...] * mp[None, :, :], axis=1)
    accs[0:3, 64:128] += jnp.sum(linap_ref[...] * ap[None, :, :], axis=1)

    @pl.when(c == pl.num_programs(0) - 1)
    def _():
        hn = hn_ref[...]
        prod = linhn_ref[...] * hn[None, :, :]
        s2 = jnp.sum(prod, axis=2)
        hnpart = jnp.sum(s2, axis=1, keepdims=True)
        logits = (jnp.sum(accs[0:3, :], axis=1, keepdims=True)
                  + hnpart + linb_ref[...])
        m = jnp.max(logits, axis=0, keepdims=True)
        e = jnp.exp(logits - m)
        out_ref[0:3, 0:1] = e / jnp.sum(e, axis=0, keepdims=True)


def _head(y2, hn, lin_W, lin_b, C):
    n = y2.shape[0]
    nc = n // C
    P = _HID // 2
    linmp = lin_W[:, :n * P].reshape(3, n, P)
    linap = lin_W[:, n * P:2 * n * P].reshape(3, n, P)
    linhn = lin_W[:, 2 * n * P:].reshape(3, 3, _HID)
    body = functools.partial(_head_body, C)
    out = pl.pallas_call(
        body,
        grid=(nc,),
        in_specs=[
            pl.BlockSpec((C, _HID), lambda c: (c, 0)),
            pl.BlockSpec((3, _HID), lambda c: (0, 0)),
            pl.BlockSpec((3, C, P), lambda c: (0, c, 0)),
            pl.BlockSpec((3, C, P), lambda c: (0, c, 0)),
            pl.BlockSpec((3, 3, _HID), lambda c: (0, 0, 0)),
            pl.BlockSpec((3, 1), lambda c: (0, 0)),
        ],
        out_specs=pl.BlockSpec((8, 128), lambda c: (0, 0)),
        out_shape=jax.ShapeDtypeStruct((8, 128), jnp.float32),
        scratch_shapes=[pltpu.VMEM((8, 128), jnp.float32)],
    )(y2, hn, linmp, linap, linhn, lin_b.reshape(3, 1))
    return out[0:3, 0]


def _gcn_jax(x, edge_index, edge_attr, W1, W2, bias):
    n = x.shape[0]
    row, col = edge_index[0], edge_index[1]
    ones_e = jnp.ones(row.shape[0], dtype=x.dtype)
    deg = jax.ops.segment_sum(ones_e, col, num_segments=n) + 1.0
    dis = deg ** -0.5
    xw = x @ W1.T
    ew = edge_attr @ W2.T
    norm = dis[row] * dis[col]
    msg = norm[:, None] * jnp.tanh(xw[row] * ew)
    agg = jax.ops.segment_sum(msg, col, num_segments=n)
    sw2 = W2.sum(axis=1)
    selfmsg = (dis * dis)[:, None] * jnp.tanh(xw * sw2[None, :])
    out = (agg + selfmsg) / deg[:, None]
    return jax.nn.sigmoid(out + bias)


def kernel(x, edge_index, edge_attr, W1, W2, bias, Wih0, Whh0, Wih1, Whh1,
           Wih2, Whh2, lin_W, lin_b):
    g = _gcn_jax(x, edge_index, edge_attr, W1, W2, bias)
    wihT = jnp.stack([Wih0.T, Wih1.T, Wih2.T])
    whhT = jnp.stack([Whh0.T, Whh1.T, Whh2.T])
    y2, hn = _lstm3(g, wihT, whhT, C=500)
    return _head(y2, hn, lin_W, lin_b, C=500)


# speedup vs baseline: 3.0059x; 3.0059x over previous
import functools

import jax
import jax.numpy as jnp
import numpy as np
from jax.experimental import pallas as pl
from jax.experimental.pallas import tpu as pltpu

_HID = 128
_G4 = 4 * _HID


def _lstm3_body(C, g_ref, wihT_ref, whhT_ref, y_ref, hn_ref, yscr, gxscr, hcscr):
    l = pl.program_id(0)
    c = pl.program_id(1)
    base = c * C

    xin = jnp.where(l == 0, g_ref[...], yscr[pl.ds(base, C), :])
    gxscr[...] = jnp.dot(xin, wihT_ref[0], preferred_element_type=jnp.float32)

    @pl.when(c == 0)
    def _():
        hcscr[...] = jnp.zeros_like(hcscr)

    whh = whhT_ref[0]

    def step(t, carry):
        h, cc = carry
        gates = gxscr[pl.ds(t, 1), :] + jnp.dot(h, whh,
                                                preferred_element_type=jnp.float32)
        i = jax.nn.sigmoid(gates[:, 0:_HID])
        f = jax.nn.sigmoid(gates[:, _HID:2 * _HID])
        gg = jnp.tanh(gates[:, 2 * _HID:3 * _HID])
        o = jax.nn.sigmoid(gates[:, 3 * _HID:4 * _HID])
        cc = f * cc + i * gg
        h = o * jnp.tanh(cc)
        yscr[pl.ds(base + t, 1), :] = h
        return (h, cc)

    h, cc = jax.lax.fori_loop(0, C, step, (hcscr[0:1, :], hcscr[1:2, :]))
    hcscr[0:1, :] = h
    hcscr[1:2, :] = cc
    y_ref[...] = yscr[pl.ds(base, C), :]

    @pl.when(c == pl.num_programs(1) - 1)
    def _():
        hn_ref[pl.ds(l, 1), :] = h


def _lstm3(g, wihT, whhT, C):
    n = g.shape[0]
    nc = n // C
    body = functools.partial(_lstm3_body, C)
    y2, hn = pl.pallas_call(
        body,
        grid=(3, nc),
        in_specs=[
            pl.BlockSpec((C, _HID), lambda l, c: (c, 0)),
            pl.BlockSpec((1, _HID, _G4), lambda l, c: (l, 0, 0)),
            pl.BlockSpec((1, _HID, _G4), lambda l, c: (l, 0, 0)),
        ],
        out_specs=[
            pl.BlockSpec((C, _HID), lambda l, c: (c, 0)),
            pl.BlockSpec((8, _HID), lambda l, c: (0, 0)),
        ],
        out_shape=[
            jax.ShapeDtypeStruct((n, _HID), jnp.float32),
            jax.ShapeDtypeStruct((8, _HID), jnp.float32),
        ],
        scratch_shapes=[
            pltpu.VMEM((n, _HID), jnp.float32),
            pltpu.VMEM((C, _G4), jnp.float32),
            pltpu.VMEM((8, _HID), jnp.float32),
        ],
    )(g, wihT, whhT)
    return y2, hn[0:3]


def _head_body(C, y2_ref, hn_ref, linmp_ref, linap_ref, linhn_ref, linb_ref,
               out_ref, accs):
    c = pl.program_id(0)

    @pl.when(c == 0)
    def _():
        accs[...] = jnp.zeros_like(accs)

    y2 = y2_ref[...]
    rolled = pltpu.roll(y2, _HID - 1, 1)
    sel = jax.lax.broadcasted_iota(jnp.int32, (C, _HID), 1) % 2 == 0
    smat = (jax.lax.broadcasted_iota(jnp.int32, (_HID, _HID // 2), 0) ==
            2 * jax.lax.broadcasted_iota(jnp.int32, (_HID, _HID // 2), 1)
            ).astype(jnp.float32)
    pairmax = jnp.maximum(y2, rolled)
    pairsum = y2 + rolled
    mp = jnp.dot(jnp.where(sel, pairmax, 0.0), smat,
                 preferred_element_type=jnp.float32)
    ap = 0.5 * jnp.dot(jnp.where(sel, pairsum, 0.0), smat,
                       preferred_element_type=jnp.float32)

    accs[0:3, 0:64] += jnp.sum(linmp_ref[...] * mp[None, :, :], axis=1)
    accs[0:3, 64:128] += jnp.sum(linap_ref[...] * ap[None, :, :], axis=1)

    @pl.when(c == pl.num_programs(0) - 1)
    def _():
        hn = hn_ref[...]
        prod = linhn_ref[...] * hn[None, :, :]
        s2 = jnp.sum(prod, axis=2)
        hnpart = jnp.sum(s2, axis=1, keepdims=True)
        logits = (jnp.sum(accs[0:3, :], axis=1, keepdims=True)
                  + hnpart + linb_ref[...])
        m = jnp.max(logits, axis=0, keepdims=True)
        e = jnp.exp(logits - m)
        out_ref[0:3, 0:1] = e / jnp.sum(e, axis=0, keepdims=True)


def _head(y2, hn, lin_W, lin_b, C):
    n = y2.shape[0]
    nc = n // C
    P = _HID // 2
    linmp = lin_W[:, :n * P].reshape(3, n, P)
    linap = lin_W[:, n * P:2 * n * P].reshape(3, n, P)
    linhn = lin_W[:, 2 * n * P:].reshape(3, 3, _HID)
    body = functools.partial(_head_body, C)
    out = pl.pallas_call(
        body,
        grid=(nc,),
        in_specs=[
            pl.BlockSpec((C, _HID), lambda c: (c, 0)),
            pl.BlockSpec((3, _HID), lambda c: (0, 0)),
            pl.BlockSpec((3, C, P), lambda c: (0, c, 0)),
            pl.BlockSpec((3, C, P), lambda c: (0, c, 0)),
            pl.BlockSpec((3, 3, _HID), lambda c: (0, 0, 0)),
            pl.BlockSpec((3, 1), lambda c: (0, 0)),
        ],
        out_specs=pl.BlockSpec((8, 128), lambda c: (0, 0)),
        out_shape=jax.ShapeDtypeStruct((8, 128), jnp.float32),
        scratch_shapes=[pltpu.VMEM((8, 128), jnp.float32)],
    )(y2, hn, linmp, linap, linhn, lin_b.reshape(3, 1))
    return out[0:3, 0]


def _gcn_jax(x, edge_index, edge_attr, W1, W2, bias):
    n = x.shape[0]
    row, col = edge_index[0], edge_index[1]
    ones_e = jnp.ones(row.shape[0], dtype=x.dtype)
    deg = jax.ops.segment_sum(ones_e, col, num_segments=n) + 1.0
    dis = deg ** -0.5
    xw = x @ W1.T
    ew = edge_attr @ W2.T
    norm = dis[row] * dis[col]
    msg = norm[:, None] * jnp.tanh(xw[row] * ew)
    agg = jax.ops.segment_sum(msg, col, num_segments=n)
    sw2 = W2.sum(axis=1)
    selfmsg = (dis * dis)[:, None] * jnp.tanh(xw * sw2[None, :])
    out = (agg + selfmsg) / deg[:, None]
    return jax.nn.sigmoid(out + bias)


def kernel(x, edge_index, edge_attr, W1, W2, bias, Wih0, Whh0, Wih1, Whh1,
           Wih2, Whh2, lin_W, lin_b):
    g = _gcn_jax(x, edge_index, edge_attr, W1, W2, bias)
    wihT = jnp.stack([Wih0.T, Wih1.T, Wih2.T])
    whhT = jnp.stack([Whh0.T, Whh1.T, Whh2.T])
    y2, hn = _lstm3(g, wihT, whhT, C=400)
    return _head(y2, hn, lin_W, lin_b, C=400)


# speedup vs baseline: 3.7992x; 1.2639x over previous
import functools

import jax
import jax.numpy as jnp
import numpy as np
from jax.experimental import pallas as pl
from jax.experimental.pallas import tpu as pltpu

_HID = 128
_G4 = 4 * _HID


_GSRC = {"i": 0, "f": 1, "g": 2, "o": 3}
_GDST = {"i": 0, "f": 3, "o": 6, "g": 9}


def _pack_cols(M, W, rowblk, l):
    H = _HID
    WT = W.T
    for g in ("i", "f", "g", "o"):
        src = WT[:, _GSRC[g] * H:(_GSRC[g] + 1) * H]
        c0 = (_GDST[g] + l) * H
        M = M.at[rowblk * H:(rowblk + 1) * H, c0:c0 + H].set(src)
    return M


def _bigmats(Wih0, Whh0, Wih1, Whh1, Wih2, Whh2):
    H = _HID
    WB = jnp.zeros((3 * H, 12 * H), jnp.float32)
    WB = _pack_cols(WB, Whh0, 0, 0)
    WB = _pack_cols(WB, Wih1, 0, 1)
    WB = _pack_cols(WB, Whh1, 1, 1)
    WB = _pack_cols(WB, Wih2, 1, 2)
    WB = _pack_cols(WB, Whh2, 2, 2)
    W0 = jnp.zeros((H, 12 * H), jnp.float32)
    W0 = _pack_cols(W0, Wih0, 0, 0)
    return W0, WB


def _lstm3f_body(C, n, g_ref, w0_ref, wb_ref, y_ref, hn_ref, gxscr, hcscr):
    H = _HID
    c = pl.program_id(0)
    nc = pl.num_programs(0)
    base = c * C
    gxscr[...] = jnp.dot(g_ref[...], w0_ref[...],
                         preferred_element_type=jnp.float32)

    @pl.when(c == 0)
    def _():
        hcscr[...] = jnp.zeros_like(hcscr)

    wb = wb_ref[...]
    steps = jnp.where(c == nc - 1, C + 2, C)

    def step(i, carry):
        h_all, c_all, hs0, hs1 = carry
        gx = gxscr[pl.ds(jnp.minimum(i, C - 1), 1), :]
        zz = gx + jnp.dot(h_all, wb, preferred_element_type=jnp.float32)
        sig = jax.nn.sigmoid(zz[:, 0:9 * H])
        gg = jnp.tanh(zz[:, 9 * H:12 * H])
        ia = sig[:, 0:3 * H]
        fa = sig[:, 3 * H:6 * H]
        oa = sig[:, 6 * H:9 * H]
        c_all = fa * c_all + ia * gg
        h_new = oa * jnp.tanh(c_all)
        widx = jnp.maximum(base + i - 2, 0)
        y_ref[pl.ds(widx, 1), :] = h_new[:, 2 * H:3 * H]
        gi = base + i
        hs0 = jnp.where(gi == n - 1, h_new[:, 0:H], hs0)
        hs1 = jnp.where(gi == n, h_new[:, H:2 * H], hs1)
        return (h_new, c_all, hs0, hs1)

    h_all, c_all, hs0, hs1 = jax.lax.fori_loop(
        0, steps, step,
        (hcscr[0:1, :], hcscr[1:2, :],
         hcscr[2:3, 0:H], hcscr[3:4, 0:H]))
    hcscr[0:1, :] = h_all
    hcscr[1:2, :] = c_all
    hcscr[2:3, 0:H] = hs0
    hcscr[3:4, 0:H] = hs1

    @pl.when(c == nc - 1)
    def _():
        hn_ref[0:1, :] = hs0
        hn_ref[1:2, :] = hs1
        hn_ref[2:3, :] = h_all[:, 2 * H:3 * H]


def _lstm3f(g, W0, WB, C):
    n = g.shape[0]
    nc = n // C
    body = functools.partial(_lstm3f_body, C, n)
    y2, hn = pl.pallas_call(
        body,
        grid=(nc,),
        in_specs=[
            pl.BlockSpec((C, _HID), lambda c: (c, 0)),
            pl.BlockSpec((_HID, 12 * _HID), lambda c: (0, 0)),
            pl.BlockSpec((3 * _HID, 12 * _HID), lambda c: (0, 0)),
        ],
        out_specs=[
            pl.BlockSpec((n, _HID), lambda c: (0, 0)),
            pl.BlockSpec((8, _HID), lambda c: (0, 0)),
        ],
        out_shape=[
            jax.ShapeDtypeStruct((n, _HID), jnp.float32),
            jax.ShapeDtypeStruct((8, _HID), jnp.float32),
        ],
        scratch_shapes=[
            pltpu.VMEM((C, 12 * _HID), jnp.float32),
            pltpu.VMEM((8, 3 * _HID), jnp.float32),
        ],
    )(g, W0, WB)
    return y2, hn[0:3]


def _lstm3_body(C, g_ref, wihT_ref, whhT_ref, y_ref, hn_ref, yscr, gxscr, hcscr):
    l = pl.program_id(0)
    c = pl.program_id(1)
    base = c * C

    xin = jnp.where(l == 0, g_ref[...], yscr[pl.ds(base, C), :])
    gxscr[...] = jnp.dot(xin, wihT_ref[0], preferred_element_type=jnp.float32)

    @pl.when(c == 0)
    def _():
        hcscr[...] = jnp.zeros_like(hcscr)

    whh = whhT_ref[0]

    def step(t, carry):
        h, cc = carry
        gates = gxscr[pl.ds(t, 1), :] + jnp.dot(h, whh,
                                                preferred_element_type=jnp.float32)
        i = jax.nn.sigmoid(gates[:, 0:_HID])
        f = jax.nn.sigmoid(gates[:, _HID:2 * _HID])
        gg = jnp.tanh(gates[:, 2 * _HID:3 * _HID])
        o = jax.nn.sigmoid(gates[:, 3 * _HID:4 * _HID])
        cc = f * cc + i * gg
        h = o * jnp.tanh(cc)
        yscr[pl.ds(base + t, 1), :] = h
        return (h, cc)

    h, cc = jax.lax.fori_loop(0, C, step, (hcscr[0:1, :], hcscr[1:2, :]))
    hcscr[0:1, :] = h
    hcscr[1:2, :] = cc
    y_ref[...] = yscr[pl.ds(base, C), :]

    @pl.when(c == pl.num_programs(1) - 1)
    def _():
        hn_ref[pl.ds(l, 1), :] = h


def _lstm3(g, wihT, whhT, C):
    n = g.shape[0]
    nc = n // C
    body = functools.partial(_lstm3_body, C)
    y2, hn = pl.pallas_call(
        body,
        grid=(3, nc),
        in_specs=[
            pl.BlockSpec((C, _HID), lambda l, c: (c, 0)),
            pl.BlockSpec((1, _HID, _G4), lambda l, c: (l, 0, 0)),
            pl.BlockSpec((1, _HID, _G4), lambda l, c: (l, 0, 0)),
        ],
        out_specs=[
            pl.BlockSpec((C, _HID), lambda l, c: (c, 0)),
            pl.BlockSpec((8, _HID), lambda l, c: (0, 0)),
        ],
        out_shape=[
            jax.ShapeDtypeStruct((n, _HID), jnp.float32),
            jax.ShapeDtypeStruct((8, _HID), jnp.float32),
        ],
        scratch_shapes=[
            pltpu.VMEM((n, _HID), jnp.float32),
            pltpu.VMEM((C, _G4), jnp.float32),
            pltpu.VMEM((8, _HID), jnp.float32),
        ],
    )(g, wihT, whhT)
    return y2, hn[0:3]


def _head_body(C, y2_ref, hn_ref, linmp_ref, linap_ref, linhn_ref, linb_ref,
               out_ref, accs):
    c = pl.program_id(0)

    @pl.when(c == 0)
    def _():
        accs[...] = jnp.zeros_like(accs)

    y2 = y2_ref[...]
    rolled = pltpu.roll(y2, _HID - 1, 1)
    sel = jax.lax.broadcasted_iota(jnp.int32, (C, _HID), 1) % 2 == 0
    smat = (jax.lax.broadcasted_iota(jnp.int32, (_HID, _HID // 2), 0) ==
            2 * jax.lax.broadcasted_iota(jnp.int32, (_HID, _HID // 2), 1)
            ).astype(jnp.float32)
    pairmax = jnp.maximum(y2, rolled)
    pairsum = y2 + rolled
    mp = jnp.dot(jnp.where(sel, pairmax, 0.0), smat,
                 preferred_element_type=jnp.float32)
    ap = 0.5 * jnp.dot(jnp.where(sel, pairsum, 0.0), smat,
                       preferred_element_type=jnp.float32)

    accs[0:3, 0:64] += jnp.sum(linmp_ref[...] * mp[None, :, :], axis=1)
    accs[0:3, 64:128] += jnp.sum(linap_ref[...] * ap[None, :, :], axis=1)

    @pl.when(c == pl.num_programs(0) - 1)
    def _():
        hn = hn_ref[...]
        prod = linhn_ref[...] * hn[None, :, :]
        s2 = jnp.sum(prod, axis=2)
        hnpart = jnp.sum(s2, axis=1, keepdims=True)
        logits = (jnp.sum(accs[0:3, :], axis=1, keepdims=True)
                  + hnpart + linb_ref[...])
        m = jnp.max(logits, axis=0, keepdims=True)
        e = jnp.exp(logits - m)
        out_ref[0:3, 0:1] = e / jnp.sum(e, axis=0, keepdims=True)


def _head(y2, hn, lin_W, lin_b, C):
    n = y2.shape[0]
    nc = n // C
    P = _HID // 2
    linmp = lin_W[:, :n * P].reshape(3, n, P)
    linap = lin_W[:, n * P:2 * n * P].reshape(3, n, P)
    linhn = lin_W[:, 2 * n * P:].reshape(3, 3, _HID)
    body = functools.partial(_head_body, C)
    out = pl.pallas_call(
        body,
        grid=(nc,),
        in_specs=[
            pl.BlockSpec((C, _HID), lambda c: (c, 0)),
            pl.BlockSpec((3, _HID), lambda c: (0, 0)),
            pl.BlockSpec((3, C, P), lambda c: (0, c, 0)),
            pl.BlockSpec((3, C, P), lambda c: (0, c, 0)),
            pl.BlockSpec((3, 3, _HID), lambda c: (0, 0, 0)),
            pl.BlockSpec((3, 1), lambda c: (0, 0)),
        ],
        out_specs=pl.BlockSpec((8, 128), lambda c: (0, 0)),
        out_shape=jax.ShapeDtypeStruct((8, 128), jnp.float32),
        scratch_shapes=[pltpu.VMEM((8, 128), jnp.float32)],
    )(y2, hn, linmp, linap, linhn, lin_b.reshape(3, 1))
    return out[0:3, 0]


def _gcn_jax(x, edge_index, edge_attr, W1, W2, bias):
    n = x.shape[0]
    row, col = edge_index[0], edge_index[1]
    ones_e = jnp.ones(row.shape[0], dtype=x.dtype)
    deg = jax.ops.segment_sum(ones_e, col, num_segments=n) + 1.0
    dis = deg ** -0.5
    xw = x @ W1.T
    ew = edge_attr @ W2.T
    norm = dis[row] * dis[col]
    msg = norm[:, None] * jnp.tanh(xw[row] * ew)
    agg = jax.ops.segment_sum(msg, col, num_segments=n)
    sw2 = W2.sum(axis=1)
    selfmsg = (dis * dis)[:, None] * jnp.tanh(xw * sw2[None, :])
    out = (agg + selfmsg) / deg[:, None]
    return jax.nn.sigmoid(out + bias)


def kernel(x, edge_index, edge_attr, W1, W2, bias, Wih0, Whh0, Wih1, Whh1,
           Wih2, Whh2, lin_W, lin_b):
    g = _gcn_jax(x, edge_index, edge_attr, W1, W2, bias)
    W0, WB = _bigmats(Wih0, Whh0, Wih1, Whh1, Wih2, Whh2)
    y2, hn = _lstm3f(g, W0, WB, C=400)
    return _head(y2, hn, lin_W, lin_b, C=400)


# speedup vs baseline: 9.1275x; 2.4025x over previous
import functools

import jax
import jax.numpy as jnp
import numpy as np
from jax import lax
from jax.experimental import pallas as pl
from jax.experimental.pallas import tpu as pltpu
from jax.experimental.pallas import tpu_sc as plsc

_HID = 128
_G4 = 4 * _HID
_N = 10000
_E = 320000
_NW = 32
_EPW = _E // _NW


_GSRC = {"i": 0, "f": 1, "g": 2, "o": 3}
_GDST = {"i": 0, "f": 3, "o": 6, "g": 9}


def _pack_cols(M, W, rowblk, l):
    H = _HID
    WT = W.T
    for g in ("i", "f", "g", "o"):
        src = WT[:, _GSRC[g] * H:(_GSRC[g] + 1) * H]
        c0 = (_GDST[g] + l) * H
        M = M.at[rowblk * H:(rowblk + 1) * H, c0:c0 + H].set(src)
    return M


def _bigmats(Wih0, Whh0, Wih1, Whh1, Wih2, Whh2):
    H = _HID
    WB = jnp.zeros((3 * H, 12 * H), jnp.float32)
    WB = _pack_cols(WB, Whh0, 0, 0)
    WB = _pack_cols(WB, Wih1, 0, 1)
    WB = _pack_cols(WB, Whh1, 1, 1)
    WB = _pack_cols(WB, Wih2, 1, 2)
    WB = _pack_cols(WB, Whh2, 2, 2)
    W0 = jnp.zeros((H, 12 * H), jnp.float32)
    W0 = _pack_cols(W0, Wih0, 0, 0)
    return W0, WB


def _lstm3f_body(C, n, g_ref, w0_ref, wb_ref, y_ref, hn_ref, gxscr, hcscr):
    H = _HID
    c = pl.program_id(0)
    nc = pl.num_programs(0)
    base = c * C
    gxscr[...] = jnp.dot(g_ref[...], w0_ref[...],
                         preferred_element_type=jnp.float32)

    @pl.when(c == 0)
    def _():
        hcscr[...] = jnp.zeros_like(hcscr)

    wb = wb_ref[...]
    steps = jnp.where(c == nc - 1, C + 2, C)

    def step(i, carry):
        h_all, c_all, hs0, hs1 = carry
        gx = gxscr[pl.ds(jnp.minimum(i, C - 1), 1), :]
        zz = gx + jnp.dot(h_all, wb, preferred_element_type=jnp.float32)
        sig = jax.nn.sigmoid(zz[:, 0:9 * H])
        gg = jnp.tanh(zz[:, 9 * H:12 * H])
        ia = sig[:, 0:3 * H]
        fa = sig[:, 3 * H:6 * H]
        oa = sig[:, 6 * H:9 * H]
        c_all = fa * c_all + ia * gg
        h_new = oa * jnp.tanh(c_all)
        widx = jnp.maximum(base + i - 2, 0)
        y_ref[pl.ds(widx, 1), :] = h_new[:, 2 * H:3 * H]
        gi = base + i
        hs0 = jnp.where(gi == n - 1, h_new[:, 0:H], hs0)
        hs1 = jnp.where(gi == n, h_new[:, H:2 * H], hs1)
        return (h_new, c_all, hs0, hs1)

    h_all, c_all, hs0, hs1 = jax.lax.fori_loop(
        0, steps, step,
        (hcscr[0:1, :], hcscr[1:2, :],
         hcscr[2:3, 0:H], hcscr[3:4, 0:H]))
    hcscr[0:1, :] = h_all
    hcscr[1:2, :] = c_all
    hcscr[2:3, 0:H] = hs0
    hcscr[3:4, 0:H] = hs1

    @pl.when(c == nc - 1)
    def _():
        hn_ref[0:1, :] = hs0
        hn_ref[1:2, :] = hs1
        hn_ref[2:3, :] = h_all[:, 2 * H:3 * H]


def _lstm3f(g, W0, WB, C):
    n = g.shape[0]
    nc = n // C
    body = functools.partial(_lstm3f_body, C, n)
    y2, hn = pl.pallas_call(
        body,
        grid=(nc,),
        in_specs=[
            pl.BlockSpec((C, _HID), lambda c: (c, 0)),
            pl.BlockSpec((_HID, 12 * _HID), lambda c: (0, 0)),
            pl.BlockSpec((3 * _HID, 12 * _HID), lambda c: (0, 0)),
        ],
        out_specs=[
            pl.BlockSpec((n, _HID), lambda c: (0, 0)),
            pl.BlockSpec((8, _HID), lambda c: (0, 0)),
        ],
        out_shape=[
            jax.ShapeDtypeStruct((n, _HID), jnp.float32),
            jax.ShapeDtypeStruct((8, _HID), jnp.float32),
        ],
        scratch_shapes=[
            pltpu.VMEM((C, 12 * _HID), jnp.float32),
            pltpu.VMEM((8, 3 * _HID), jnp.float32),
        ],
    )(g, W0, WB)
    return y2, hn[0:3]


def _lstm3_body(C, g_ref, wihT_ref, whhT_ref, y_ref, hn_ref, yscr, gxscr, hcscr):
    l = pl.program_id(0)
    c = pl.program_id(1)
    base = c * C

    xin = jnp.where(l == 0, g_ref[...], yscr[pl.ds(base, C), :])
    gxscr[...] = jnp.dot(xin, wihT_ref[0], preferred_element_type=jnp.float32)

    @pl.when(c == 0)
    def _():
        hcscr[...] = jnp.zeros_like(hcscr)

    whh = whhT_ref[0]

    def step(t, carry):
        h, cc = carry
        gates = gxscr[pl.ds(t, 1), :] + jnp.dot(h, whh,
                                                preferred_element_type=jnp.float32)
        i = jax.nn.sigmoid(gates[:, 0:_HID])
        f = jax.nn.sigmoid(gates[:, _HID:2 * _HID])
        gg = jnp.tanh(gates[:, 2 * _HID:3 * _HID])
        o = jax.nn.sigmoid(gates[:, 3 * _HID:4 * _HID])
        cc = f * cc + i * gg
        h = o * jnp.tanh(cc)
        yscr[pl.ds(base + t, 1), :] = h
        return (h, cc)

    h, cc = jax.lax.fori_loop(0, C, step, (hcscr[0:1, :], hcscr[1:2, :]))
    hcscr[0:1, :] = h
    hcscr[1:2, :] = cc
    y_ref[...] = yscr[pl.ds(base, C), :]

    @pl.when(c == pl.num_programs(1) - 1)
    def _():
        hn_ref[pl.ds(l, 1), :] = h


def _lstm3(g, wihT, whhT, C):
    n = g.shape[0]
    nc = n // C
    body = functools.partial(_lstm3_body, C)
    y2, hn = pl.pallas_call(
        body,
        grid=(3, nc),
        in_specs=[
            pl.BlockSpec((C, _HID), lambda l, c: (c, 0)),
            pl.BlockSpec((1, _HID, _G4), lambda l, c: (l, 0, 0)),
            pl.BlockSpec((1, _HID, _G4), lambda l, c: (l, 0, 0)),
        ],
        out_specs=[
            pl.BlockSpec((C, _HID), lambda l, c: (c, 0)),
            pl.BlockSpec((8, _HID), lambda l, c: (0, 0)),
        ],
        out_shape=[
            jax.ShapeDtypeStruct((n, _HID), jnp.float32),
            jax.ShapeDtypeStruct((8, _HID), jnp.float32),
        ],
        scratch_shapes=[
            pltpu.VMEM((n, _HID), jnp.float32),
            pltpu.VMEM((C, _G4), jnp.float32),
            pltpu.VMEM((8, _HID), jnp.float32),
        ],
    )(g, wihT, whhT)
    return y2, hn[0:3]


def _head_body(C, y2_ref, hn_ref, linmp_ref, linap_ref, linhn_ref, linb_ref,
               out_ref, accs):
    c = pl.program_id(0)

    @pl.when(c == 0)
    def _():
        accs[...] = jnp.zeros_like(accs)

    y2 = y2_ref[...]
    rolled = pltpu.roll(y2, _HID - 1, 1)
    sel = jax.lax.broadcasted_iota(jnp.int32, (C, _HID), 1) % 2 == 0
    smat = (jax.lax.broadcasted_iota(jnp.int32, (_HID, _HID // 2), 0) ==
            2 * jax.lax.broadcasted_iota(jnp.int32, (_HID, _HID // 2), 1)
            ).astype(jnp.float32)
    pairmax = jnp.maximum(y2, rolled)
    pairsum = y2 + rolled
    mp = jnp.dot(jnp.where(sel, pairmax, 0.0), smat,
                 preferred_element_type=jnp.float32)
    ap = 0.5 * jnp.dot(jnp.where(sel, pairsum, 0.0), smat,
                       preferred_element_type=jnp.float32)

    accs[0:3, 0:64] += jnp.sum(linmp_ref[...] * mp[None, :, :], axis=1)
    accs[0:3, 64:128] += jnp.sum(linap_ref[...] * ap[None, :, :], axis=1)

    @pl.when(c == pl.num_programs(0) - 1)
    def _():
        hn = hn_ref[...]
        prod = linhn_ref[...] * hn[None, :, :]
        s2 = jnp.sum(prod, axis=2)
        hnpart = jnp.sum(s2, axis=1, keepdims=True)
        logits = (jnp.sum(accs[0:3, :], axis=1, keepdims=True)
                  + hnpart + linb_ref[...])
        m = jnp.max(logits, axis=0, keepdims=True)
        e = jnp.exp(logits - m)
        out_ref[0:3, 0:1] = e / jnp.sum(e, axis=0, keepdims=True)


def _head(y2, hn, lin_W, lin_b, C):
    n = y2.shape[0]
    nc = n // C
    P = _HID // 2
    linmp = lin_W[:, :n * P].reshape(3, n, P)
    linap = lin_W[:, n * P:2 * n * P].reshape(3, n, P)
    linhn = lin_W[:, 2 * n * P:].reshape(3, 3, _HID)
    body = functools.partial(_head_body, C)
    out = pl.pallas_call(
        body,
        grid=(nc,),
        in_specs=[
            pl.BlockSpec((C, _HID), lambda c: (c, 0)),
            pl.BlockSpec((3, _HID), lambda c: (0, 0)),
            pl.BlockSpec((3, C, P), lambda c: (0, c, 0)),
            pl.BlockSpec((3, C, P), lambda c: (0, c, 0)),
            pl.BlockSpec((3, 3, _HID), lambda c: (0, 0, 0)),
            pl.BlockSpec((3, 1), lambda c: (0, 0)),
        ],
        out_specs=pl.BlockSpec((8, 128), lambda c: (0, 0)),
        out_shape=jax.ShapeDtypeStruct((8, 128), jnp.float32),
        scratch_shapes=[pltpu.VMEM((8, 128), jnp.float32)],
    )(y2, hn, linmp, linap, linhn, lin_b.reshape(3, 1))
    return out[0:3, 0]


def _sc_hist(col):
    mesh = plsc.VectorSubcoreMesh(core_axis_name="c", subcore_axis_name="s")
    CH = 2000

    @functools.partial(
        pl.kernel, mesh=mesh,
        compiler_params=pltpu.CompilerParams(needs_layout_passes=False),
        out_type=jax.ShapeDtypeStruct((_NW, _N), jnp.float32),
        scratch_types=[
            pltpu.VMEM((CH,), jnp.int32),
            pltpu.VMEM((_N,), jnp.float32),
        ],
    )
    def k(col_hbm, out_hbm, colv, hist):
        cid = lax.axis_index("c")
        sid = lax.axis_index("s")
        wid = cid * 16 + sid
        base = wid * _EPW
        zeros = jnp.zeros((16,), jnp.float32)
        ones = jnp.ones((16,), jnp.float32)

        def zloop(i, _):
            hist[pl.ds(i * 16, 16)] = zeros
            return 0
        lax.fori_loop(0, _N // 16, zloop, 0)

        def chunk(cb, _):
            pltpu.sync_copy(col_hbm.at[pl.ds(base + cb * CH, CH)], colv)

            def inner(j, _):
                idx = colv[pl.ds(j * 16, 16)]
                plsc.addupdate_scatter(hist, [idx], ones)
                return 0
            lax.fori_loop(0, CH // 16, inner, 0)
            return 0
        lax.fori_loop(0, _EPW // CH, chunk, 0)
        pltpu.sync_copy(hist, out_hbm.at[wid])

    return k(col)


def _sc_msgs(row, col, dis, xw, ew):
    mesh = plsc.VectorSubcoreMesh(core_axis_name="c", subcore_axis_name="s")
    B = 80
    NB = _EPW // B
    NP = 10240
    SR = NP // 16
    ZR = 128

    @functools.partial(
        pl.kernel, mesh=mesh,
        compiler_params=pltpu.CompilerParams(needs_layout_passes=False),
        out_type=jax.ShapeDtypeStruct((2, NP, _HID), jnp.float32),
        scratch_types=[
            pltpu.VMEM((_N,), jnp.float32),
            pltpu.VMEM((B,), jnp.int32),
            pltpu.VMEM((B,), jnp.int32),
            pltpu.VMEM((B + 16,), jnp.float32),
            pltpu.VMEM((B, _HID), jnp.float32),
            pltpu.VMEM((B, _HID), jnp.float32),
            pltpu.VMEM((ZR, _HID), jnp.float32),
            pltpu.VMEM_SHARED((NP, _HID), jnp.float32),
            pltpu.SemaphoreType.DMA,
        ],
    )
    def k(row_hbm, col_hbm, dis_hbm, xw_hbm, ew_hbm, out_hbm,
          disv, rowi, coli, normv, ewb, xgb, zb, agg, sem):
        cid = lax.axis_index("c")
        sid = lax.axis_index("s")
        wid = cid * 16 + sid
        base = wid * _EPW
        pltpu.sync_copy(dis_hbm, disv)

        zeros = jnp.zeros((16,), jnp.float32)

        def zrow(i, _):
            for kk in range(_HID // 16):
                zb[i, pl.ds(kk * 16, 16)] = zeros
            return 0
        lax.fori_loop(0, ZR, zrow, 0)
        r0 = sid * SR
        for z in range(SR // ZR):
            pltpu.sync_copy(zb, agg.at[pl.ds(r0 + z * ZR, ZR)])
        plsc.subcore_barrier()

        def blk(b, _):
            e0 = base + b * B
            pltpu.sync_copy(row_hbm.at[pl.ds(e0, B)], rowi)
            pltpu.sync_copy(col_hbm.at[pl.ds(e0, B)], coli)
            pltpu.sync_copy(ew_hbm.at[pl.ds(e0, B)], ewb)
            pltpu.async_copy(xw_hbm.at[rowi], xgb, sem).wait()

            def nrm(j, _):
                ri = rowi[pl.ds(j * 16, 16)]
                ci = coli[pl.ds(j * 16, 16)]
                dr = plsc.load_gather(disv, [ri])
                dc = plsc.load_gather(disv, [ci])
                normv[pl.ds(j * 16, 16)] = dr * dc
                return 0
            lax.fori_loop(0, B // 16, nrm, 0)

            def edge(j, _):
                nj = normv[pl.ds(j, 16)][0]
                for kk in range(_HID // 16):
                    xv = xgb[j, pl.ds(kk * 16, 16)]
                    ev = ewb[j, pl.ds(kk * 16, 16)]
                    p = xv * ev
                    e2 = jnp.exp(p + p)
                    t = 1.0 - 2.0 / (e2 + 1.0)
                    xgb[j, pl.ds(kk * 16, 16)] = nj * t
                return 0
            lax.fori_loop(0, B, edge, 0)
            pltpu.sync_copy(xgb, agg.at[coli], add=True)
            return 0
        lax.fori_loop(0, NB, blk, 0)

        plsc.subcore_barrier()
        for z in range(SR // ZR):
            pltpu.sync_copy(agg.at[pl.ds(r0 + z * ZR, ZR)], zb)
            pltpu.sync_copy(zb, out_hbm.at[cid, pl.ds(r0 + z * ZR, ZR)])

    return k(row, col, dis, xw, ew)[:, :_N, :]


def _mm_body(x_ref, w_ref, o_ref):
    o_ref[...] = jnp.dot(x_ref[...], w_ref[...],
                         preferred_element_type=jnp.float32)


def _matmul(x, w, CB):
    n, k = x.shape
    m = w.shape[1]
    return pl.pallas_call(
        _mm_body,
        grid=(n // CB,),
        in_specs=[pl.BlockSpec((CB, k), lambda c: (c, 0)),
                  pl.BlockSpec((k, m), lambda c: (0, 0))],
        out_specs=pl.BlockSpec((CB, m), lambda c: (c, 0)),
        out_shape=jax.ShapeDtypeStruct((n, m), jnp.float32),
    )(x, w)


def _gcnD_body(hist_ref, w2t_ref, dis_ref, rdeg_ref, sfac_ref, sw2_ref):
    ones32 = jnp.ones((_NW, 1), jnp.float32)
    degT = lax.dot_general(hist_ref[...], ones32,
                           (((0,), (0,)), ((), ())),
                           preferred_element_type=jnp.float32) + 1.0
    rdeg = 1.0 / degT
    dis_ref[...] = lax.rsqrt(degT)
    rdeg_ref[...] = rdeg
    sfac_ref[...] = rdeg * rdeg
    sw2_ref[...] = jnp.sum(w2t_ref[...], axis=0, keepdims=True)


def _gcnD(hist, W2T):
    return pl.pallas_call(
        _gcnD_body,
        out_shape=[jax.ShapeDtypeStruct((_N, 1), jnp.float32),
                   jax.ShapeDtypeStruct((_N, 1), jnp.float32),
                   jax.ShapeDtypeStruct((_N, 1), jnp.float32),
                   jax.ShapeDtypeStruct((1, _HID), jnp.float32)],
    )(hist, W2T)


def _gcnB_body(agg0_ref, agg1_ref, xw_ref, rdeg_ref, sfac_ref, sw2_ref,
               bias_ref, g_ref):
    agg = agg0_ref[0] + agg1_ref[0]
    t = jnp.tanh(xw_ref[...] * sw2_ref[...])
    pre = (agg * rdeg_ref[...] + sfac_ref[...] * t + bias_ref[...])
    g_ref[...] = jax.nn.sigmoid(pre)


def _gcnB(aggp, xw, rdeg, sfac, sw2, bias, CB):
    return pl.pallas_call(
        _gcnB_body,
        grid=(_N // CB,),
        in_specs=[
            pl.BlockSpec((1, CB, _HID), lambda c: (0, c, 0)),
            pl.BlockSpec((1, CB, _HID), lambda c: (1, c, 0)),
            pl.BlockSpec((CB, _HID), lambda c: (c, 0)),
            pl.BlockSpec((CB, 1), lambda c: (c, 0)),
            pl.BlockSpec((CB, 1), lambda c: (c, 0)),
            pl.BlockSpec((1, _HID), lambda c: (0, 0)),
            pl.BlockSpec((1, _HID), lambda c: (0, 0)),
        ],
        out_specs=pl.BlockSpec((CB, _HID), lambda c: (c, 0)),
        out_shape=jax.ShapeDtypeStruct((_N, _HID), jnp.float32),
    )(aggp, aggp, xw, rdeg, sfac, sw2, bias.reshape(1, _HID))


def _gcn_sc(x, edge_index, edge_attr, W1, W2, bias):
    row = edge_index[0]
    col = edge_index[1]
    xw = _matmul(x, W1.T, CB=1000)
    ew = _matmul(edge_attr, W2.T, CB=4000)
    hist = _sc_hist(col)
    dis, rdeg, sfac, sw2 = _gcnD(hist, W2.T)
    aggp = _sc_msgs(row, col, dis.reshape(_N), xw, ew)
    return _gcnB(aggp, xw, rdeg, sfac, sw2, bias, CB=1000)


def _gcn_jax(x, edge_index, edge_attr, W1, W2, bias):
    n = x.shape[0]
    row, col = edge_index[0], edge_index[1]
    ones_e = jnp.ones(row.shape[0], dtype=x.dtype)
    deg = jax.ops.segment_sum(ones_e, col, num_segments=n) + 1.0
    dis = deg ** -0.5
    xw = x @ W1.T
    ew = edge_attr @ W2.T
    norm = dis[row] * dis[col]
    msg = norm[:, None] * jnp.tanh(xw[row] * ew)
    agg = jax.ops.segment_sum(msg, col, num_segments=n)
    sw2 = W2.sum(axis=1)
    selfmsg = (dis * dis)[:, None] * jnp.tanh(xw * sw2[None, :])
    out = (agg + selfmsg) / deg[:, None]
    return jax.nn.sigmoid(out + bias)


def kernel(x, edge_index, edge_attr, W1, W2, bias, Wih0, Whh0, Wih1, Whh1,
           Wih2, Whh2, lin_W, lin_b):
    g = _gcn_sc(x, edge_index, edge_attr, W1, W2, bias)
    W0, WB = _bigmats(Wih0, Whh0, Wih1, Whh1, Wih2, Whh2)
    y2, hn = _lstm3f(g, W0, WB, C=400)
    return _head(y2, hn, lin_W, lin_b, C=400)


# speedup vs baseline: 10.8641x; 1.1903x over previous
import functools

import jax
import jax.numpy as jnp
import numpy as np
from jax import lax
from jax.experimental import pallas as pl
from jax.experimental.pallas import tpu as pltpu
from jax.experimental.pallas import tpu_sc as plsc

_HID = 128
_G4 = 4 * _HID
_N = 10000
_E = 320000
_NW = 32
_EPW = _E // _NW


_GSRC = {"i": 0, "f": 1, "g": 2, "o": 3}
_GDST = {"i": 0, "f": 3, "o": 6, "g": 9}


def _pack_cols(M, W, rowblk, l):
    H = _HID
    WT = W.T
    for g in ("i", "f", "g", "o"):
        src = WT[:, _GSRC[g] * H:(_GSRC[g] + 1) * H]
        c0 = (_GDST[g] + l) * H
        M = M.at[rowblk * H:(rowblk + 1) * H, c0:c0 + H].set(src)
    return M


def _bigmats(Wih0, Whh0, Wih1, Whh1, Wih2, Whh2):
    H = _HID
    WB = jnp.zeros((3 * H, 12 * H), jnp.float32)
    WB = _pack_cols(WB, Whh0, 0, 0)
    WB = _pack_cols(WB, Wih1, 0, 1)
    WB = _pack_cols(WB, Whh1, 1, 1)
    WB = _pack_cols(WB, Wih2, 1, 2)
    WB = _pack_cols(WB, Whh2, 2, 2)
    W0 = jnp.zeros((H, 12 * H), jnp.float32)
    W0 = _pack_cols(W0, Wih0, 0, 0)
    return W0, WB


def _lstm3f_body(C, n, g_ref, w0_ref, wb_ref, y_ref, hn_ref, gxscr, hcscr):
    H = _HID
    c = pl.program_id(0)
    nc = pl.num_programs(0)
    base = c * C
    gxscr[...] = jnp.dot(g_ref[...], w0_ref[...],
                         preferred_element_type=jnp.float32)

    @pl.when(c == 0)
    def _():
        hcscr[...] = jnp.zeros_like(hcscr)

    wb = wb_ref[...]
    steps = jnp.where(c == nc - 1, C + 2, C)

    def step(i, carry):
        h_all, c_all, hs0, hs1 = carry
        gx = gxscr[pl.ds(jnp.minimum(i, C - 1), 1), :]
        zz = gx + jnp.dot(h_all, wb, preferred_element_type=jnp.float32)
        sig = jax.nn.sigmoid(zz[:, 0:9 * H])
        gg = jnp.tanh(zz[:, 9 * H:12 * H])
        ia = sig[:, 0:3 * H]
        fa = sig[:, 3 * H:6 * H]
        oa = sig[:, 6 * H:9 * H]
        c_all = fa * c_all + ia * gg
        h_new = oa * jnp.tanh(c_all)
        widx = jnp.maximum(base + i - 2, 0)
        y_ref[pl.ds(widx, 1), :] = h_new[:, 2 * H:3 * H]
        gi = base + i
        hs0 = jnp.where(gi == n - 1, h_new[:, 0:H], hs0)
        hs1 = jnp.where(gi == n, h_new[:, H:2 * H], hs1)
        return (h_new, c_all, hs0, hs1)

    h_all, c_all, hs0, hs1 = jax.lax.fori_loop(
        0, steps, step,
        (hcscr[0:1, :], hcscr[1:2, :],
         hcscr[2:3, 0:H], hcscr[3:4, 0:H]))
    hcscr[0:1, :] = h_all
    hcscr[1:2, :] = c_all
    hcscr[2:3, 0:H] = hs0
    hcscr[3:4, 0:H] = hs1

    @pl.when(c == nc - 1)
    def _():
        hn_ref[0:1, :] = hs0
        hn_ref[1:2, :] = hs1
        hn_ref[2:3, :] = h_all[:, 2 * H:3 * H]


def _lstm3f(g, W0, WB, C):
    n = g.shape[0]
    nc = n // C
    body = functools.partial(_lstm3f_body, C, n)
    y2, hn = pl.pallas_call(
        body,
        grid=(nc,),
        in_specs=[
            pl.BlockSpec((C, _HID), lambda c: (c, 0)),
            pl.BlockSpec((_HID, 12 * _HID), lambda c: (0, 0)),
            pl.BlockSpec((3 * _HID, 12 * _HID), lambda c: (0, 0)),
        ],
        out_specs=[
            pl.BlockSpec((n, _HID), lambda c: (0, 0)),
            pl.BlockSpec((8, _HID), lambda c: (0, 0)),
        ],
        out_shape=[
            jax.ShapeDtypeStruct((n, _HID), jnp.float32),
            jax.ShapeDtypeStruct((8, _HID), jnp.float32),
        ],
        scratch_shapes=[
            pltpu.VMEM((C, 12 * _HID), jnp.float32),
            pltpu.VMEM((8, 3 * _HID), jnp.float32),
        ],
    )(g, W0, WB)
    return y2, hn[0:3]


def _reorder_ifog(W):
    H = _HID
    WT = W.T
    return jnp.concatenate(
        [WT[:, 0:H], WT[:, H:2 * H], WT[:, 3 * H:4 * H], WT[:, 2 * H:3 * H]],
        axis=1)


def _splitmats(Wih0, Whh0, Wih1, Whh1, Wih2, Whh2):
    W0 = _reorder_ifog(Wih0)
    WA = _reorder_ifog(Whh0).astype(jnp.bfloat16)
    WB1 = jnp.concatenate([_reorder_ifog(Wih1), _reorder_ifog(Whh1)],
                          axis=0).astype(jnp.bfloat16)
    WB2 = jnp.concatenate([_reorder_ifog(Wih2), _reorder_ifog(Whh2)],
                          axis=0).astype(jnp.bfloat16)
    return W0, WA, WB1, WB2


def _mixdot(a, b):
    return lax.dot_general(a, b, (((1,), (0,)), ((), ())),
                           preferred_element_type=jnp.float32)


def _lstm3s_body(C, n, g_ref, w0_ref, wa_ref, wb1_ref, wb2_ref,
                 y_ref, hn_ref, gxscr, hcscr):
    H = _HID
    c = pl.program_id(0)
    nc = pl.num_programs(0)
    base = c * C
    gxscr[...] = jnp.dot(g_ref[...], w0_ref[...],
                         preferred_element_type=jnp.float32)

    @pl.when(c == 0)
    def _():
        hcscr[...] = jnp.zeros_like(hcscr)

    wa = wa_ref[...]
    wb1 = wb1_ref[...]
    wb2 = wb2_ref[...]
    steps = jnp.where(c == nc - 1, C + 2, C)

    def upd(zz, cc):
        sg = jax.nn.sigmoid(zz[:, 0:3 * H])
        gg = jnp.tanh(zz[:, 3 * H:4 * H])
        ccn = sg[:, H:2 * H] * cc + sg[:, 0:H] * gg
        return sg[:, 2 * H:3 * H] * jnp.tanh(ccn), ccn

    def step(i, carry):
        h0, h1, h2, c0, c1, c2, hs0, hs1 = carry
        gx = gxscr[pl.ds(jnp.minimum(i, C - 1), 1), :]
        zz0 = gx + _mixdot(h0, wa)
        zz1 = _mixdot(jnp.concatenate([h0, h1], axis=1), wb1)
        zz2 = _mixdot(jnp.concatenate([h1, h2], axis=1), wb2)
        h0n, c0 = upd(zz0, c0)
        h1n, c1 = upd(zz1, c1)
        h2n, c2 = upd(zz2, c2)
        y_ref[pl.ds(jnp.maximum(base + i - 2, 0), 1), :] = h2n
        gi = base + i
        hs0 = jnp.where(gi == n - 1, h0n, hs0)
        hs1 = jnp.where(gi == n, h1n, hs1)
        return (h0n, h1n, h2n, c0, c1, c2, hs0, hs1)

    out = jax.lax.fori_loop(
        0, steps, step,
        tuple(hcscr[pl.ds(r, 1), :] for r in range(8)))
    for r in range(8):
        hcscr[pl.ds(r, 1), :] = out[r]

    @pl.when(c == nc - 1)
    def _():
        hn_ref[0:1, :] = out[6]
        hn_ref[1:2, :] = out[7]
        hn_ref[2:3, :] = out[2]


def _lstm3s(g, W0, WA, WB1, WB2, C):
    n = g.shape[0]
    nc = n // C
    body = functools.partial(_lstm3s_body, C, n)
    y2, hn = pl.pallas_call(
        body,
        grid=(nc,),
        in_specs=[
            pl.BlockSpec((C, _HID), lambda c: (c, 0)),
            pl.BlockSpec((_HID, _G4), lambda c: (0, 0)),
            pl.BlockSpec((_HID, _G4), lambda c: (0, 0)),
            pl.BlockSpec((2 * _HID, _G4), lambda c: (0, 0)),
            pl.BlockSpec((2 * _HID, _G4), lambda c: (0, 0)),
        ],
        out_specs=[
            pl.BlockSpec((n, _HID), lambda c: (0, 0)),
            pl.BlockSpec((8, _HID), lambda c: (0, 0)),
        ],
        out_shape=[
            jax.ShapeDtypeStruct((n, _HID), jnp.float32),
            jax.ShapeDtypeStruct((8, _HID), jnp.float32),
        ],
        scratch_shapes=[
            pltpu.VMEM((C, _G4), jnp.float32),
            pltpu.VMEM((8, _HID), jnp.float32),
        ],
    )(g, W0, WA, WB1, WB2)
    return y2, hn[0:3]


def _lstm3_body(C, g_ref, wihT_ref, whhT_ref, y_ref, hn_ref, yscr, gxscr, hcscr):
    l = pl.program_id(0)
    c = pl.program_id(1)
    base = c * C

    xin = jnp.where(l == 0, g_ref[...], yscr[pl.ds(base, C), :])
    gxscr[...] = jnp.dot(xin, wihT_ref[0], preferred_element_type=jnp.float32)

    @pl.when(c == 0)
    def _():
        hcscr[...] = jnp.zeros_like(hcscr)

    whh = whhT_ref[0]

    def step(t, carry):
        h, cc = carry
        gates = gxscr[pl.ds(t, 1), :] + jnp.dot(h, whh,
                                                preferred_element_type=jnp.float32)
        i = jax.nn.sigmoid(gates[:, 0:_HID])
        f = jax.nn.sigmoid(gates[:, _HID:2 * _HID])
        gg = jnp.tanh(gates[:, 2 * _HID:3 * _HID])
        o = jax.nn.sigmoid(gates[:, 3 * _HID:4 * _HID])
        cc = f * cc + i * gg
        h = o * jnp.tanh(cc)
        yscr[pl.ds(base + t, 1), :] = h
        return (h, cc)

    h, cc = jax.lax.fori_loop(0, C, step, (hcscr[0:1, :], hcscr[1:2, :]))
    hcscr[0:1, :] = h
    hcscr[1:2, :] = cc
    y_ref[...] = yscr[pl.ds(base, C), :]

    @pl.when(c == pl.num_programs(1) - 1)
    def _():
        hn_ref[pl.ds(l, 1), :] = h


def _lstm3(g, wihT, whhT, C):
    n = g.shape[0]
    nc = n // C
    body = functools.partial(_lstm3_body, C)
    y2, hn = pl.pallas_call(
        body,
        grid=(3, nc),
        in_specs=[
            pl.BlockSpec((C, _HID), lambda l, c: (c, 0)),
            pl.BlockSpec((1, _HID, _G4), lambda l, c: (l, 0, 0)),
            pl.BlockSpec((1, _HID, _G4), lambda l, c: (l, 0, 0)),
        ],
        out_specs=[
            pl.BlockSpec((C, _HID), lambda l, c: (c, 0)),
            pl.BlockSpec((8, _HID), lambda l, c: (0, 0)),
        ],
        out_shape=[
            jax.ShapeDtypeStruct((n, _HID), jnp.float32),
            jax.ShapeDtypeStruct((8, _HID), jnp.float32),
        ],
        scratch_shapes=[
            pltpu.VMEM((n, _HID), jnp.float32),
            pltpu.VMEM((C, _G4), jnp.float32),
            pltpu.VMEM((8, _HID), jnp.float32),
        ],
    )(g, wihT, whhT)
    return y2, hn[0:3]


def _head_body(C, y2_ref, hn_ref, linmp_ref, linap_ref, linhn_ref, linb_ref,
               out_ref, accs):
    c = pl.program_id(0)

    @pl.when(c == 0)
    def _():
        accs[...] = jnp.zeros_like(accs)

    y2 = y2_ref[...]
    rolled = pltpu.roll(y2, _HID - 1, 1)
    sel = jax.lax.broadcasted_iota(jnp.int32, (C, _HID), 1) % 2 == 0
    smat = (jax.lax.broadcasted_iota(jnp.int32, (_HID, _HID // 2), 0) ==
            2 * jax.lax.broadcasted_iota(jnp.int32, (_HID, _HID // 2), 1)
            ).astype(jnp.float32)
    pairmax = jnp.maximum(y2, rolled)
    pairsum = y2 + rolled
    mp = jnp.dot(jnp.where(sel, pairmax, 0.0), smat,
                 preferred_element_type=jnp.float32)
    ap = 0.5 * jnp.dot(jnp.where(sel, pairsum, 0.0), smat,
                       preferred_element_type=jnp.float32)

    accs[0:3, 0:64] += jnp.sum(linmp_ref[...] * mp[None, :, :], axis=1)
    accs[0:3, 64:128] += jnp.sum(linap_ref[...] * ap[None, :, :], axis=1)

    @pl.when(c == pl.num_programs(0) - 1)
    def _():
        hn = hn_ref[...]
        prod = linhn_ref[...] * hn[None, :, :]
        s2 = jnp.sum(prod, axis=2)
        hnpart = jnp.sum(s2, axis=1, keepdims=True)
        logits = (jnp.sum(accs[0:3, :], axis=1, keepdims=True)
                  + hnpart + linb_ref[...])
        m = jnp.max(logits, axis=0, keepdims=True)
        e = jnp.exp(logits - m)
        out_ref[0:3, 0:1] = e / jnp.sum(e, axis=0, keepdims=True)


def _head(y2, hn, lin_W, lin_b, C):
    n = y2.shape[0]
    nc = n // C
    P = _HID // 2
    linmp = lin_W[:, :n * P].reshape(3, n, P)
    linap = lin_W[:, n * P:2 * n * P].reshape(3, n, P)
    linhn = lin_W[:, 2 * n * P:].reshape(3, 3, _HID)
    body = functools.partial(_head_body, C)
    out = pl.pallas_call(
        body,
        grid=(nc,),
        in_specs=[
            pl.BlockSpec((C, _HID), lambda c: (c, 0)),
            pl.BlockSpec((3, _HID), lambda c: (0, 0)),
            pl.BlockSpec((3, C, P), lambda c: (0, c, 0)),
            pl.BlockSpec((3, C, P), lambda c: (0, c, 0)),
            pl.BlockSpec((3, 3, _HID), lambda c: (0, 0, 0)),
            pl.BlockSpec((3, 1), lambda c: (0, 0)),
        ],
        out_specs=pl.BlockSpec((8, 128), lambda c: (0, 0)),
        out_shape=jax.ShapeDtypeStruct((8, 128), jnp.float32),
        scratch_shapes=[pltpu.VMEM((8, 128), jnp.float32)],
    )(y2, hn, linmp, linap, linhn, lin_b.reshape(3, 1))
    return out[0:3, 0]


def _sc_hist(col):
    mesh = plsc.VectorSubcoreMesh(core_axis_name="c", subcore_axis_name="s")
    CH = 2000

    @functools.partial(
        pl.kernel, mesh=mesh,
        compiler_params=pltpu.CompilerParams(needs_layout_passes=False),
        out_type=jax.ShapeDtypeStruct((_NW, _N), jnp.float32),
        scratch_types=[
            pltpu.VMEM((CH,), jnp.int32),
            pltpu.VMEM((_N,), jnp.float32),
        ],
    )
    def k(col_hbm, out_hbm, colv, hist):
        cid = lax.axis_index("c")
        sid = lax.axis_index("s")
        wid = cid * 16 + sid
        base = wid * _EPW
        zeros = jnp.zeros((16,), jnp.float32)
        ones = jnp.ones((16,), jnp.float32)

        def zloop(i, _):
            hist[pl.ds(i * 16, 16)] = zeros
            return 0
        lax.fori_loop(0, _N // 16, zloop, 0)

        def chunk(cb, _):
            pltpu.sync_copy(col_hbm.at[pl.ds(base + cb * CH, CH)], colv)

            def inner(j, _):
                idx = colv[pl.ds(j * 16, 16)]
                plsc.addupdate_scatter(hist, [idx], ones)
                return 0
            lax.fori_loop(0, CH // 16, inner, 0)
            return 0
        lax.fori_loop(0, _EPW // CH, chunk, 0)
        pltpu.sync_copy(hist, out_hbm.at[wid])

    return k(col)


def _sc_msgs(row, col, dis, xw, ew):
    mesh = plsc.VectorSubcoreMesh(core_axis_name="c", subcore_axis_name="s")
    B = 80
    NB = _EPW // B
    NP = 10240
    SR = NP // 16
    ZR = 128

    @functools.partial(
        pl.kernel, mesh=mesh,
        compiler_params=pltpu.CompilerParams(needs_layout_passes=False),
        out_type=jax.ShapeDtypeStruct((2, NP, _HID), jnp.float32),
        scratch_types=[
            pltpu.VMEM((_N,), jnp.float32),
            pltpu.VMEM((B,), jnp.int32),
            pltpu.VMEM((B,), jnp.int32),
            pltpu.VMEM((B + 16,), jnp.float32),
            pltpu.VMEM((B, _HID), jnp.float32),
            pltpu.VMEM((B, _HID), jnp.float32),
            pltpu.VMEM((ZR, _HID), jnp.float32),
            pltpu.VMEM_SHARED((NP, _HID), jnp.float32),
            pltpu.SemaphoreType.DMA,
        ],
    )
    def k(row_hbm, col_hbm, dis_hbm, xw_hbm, ew_hbm, out_hbm,
          disv, rowi, coli, normv, ewb, xgb, zb, agg, sem):
        cid = lax.axis_index("c")
        sid = lax.axis_index("s")
        wid = cid * 16 + sid
        base = wid * _EPW
        pltpu.sync_copy(dis_hbm, disv)

        zeros = jnp.zeros((16,), jnp.float32)

        def zrow(i, _):
            for kk in range(_HID // 16):
                zb[i, pl.ds(kk * 16, 16)] = zeros
            return 0
        lax.fori_loop(0, ZR, zrow, 0)
        r0 = sid * SR
        for z in range(SR // ZR):
            pltpu.sync_copy(zb, agg.at[pl.ds(r0 + z * ZR, ZR)])
        plsc.subcore_barrier()

        def blk(b, _):
            e0 = base + b * B
            pltpu.sync_copy(row_hbm.at[pl.ds(e0, B)], rowi)
            pltpu.sync_copy(col_hbm.at[pl.ds(e0, B)], coli)
            pltpu.sync_copy(ew_hbm.at[pl.ds(e0, B)], ewb)
            pltpu.async_copy(xw_hbm.at[rowi], xgb, sem).wait()

            def nrm(j, _):
                ri = rowi[pl.ds(j * 16, 16)]
                ci = coli[pl.ds(j * 16, 16)]
                dr = plsc.load_gather(disv, [ri])
                dc = plsc.load_gather(disv, [ci])
                normv[pl.ds(j * 16, 16)] = dr * dc
                return 0
            lax.fori_loop(0, B // 16, nrm, 0)

            def edge(j, _):
                nj = normv[pl.ds(j, 16)][0]
                for kk in range(_HID // 16):
                    xv = xgb[j, pl.ds(kk * 16, 16)]
                    ev = ewb[j, pl.ds(kk * 16, 16)]
                    p = xv * ev
                    e2 = jnp.exp(p + p)
                    t = 1.0 - 2.0 / (e2 + 1.0)
                    xgb[j, pl.ds(kk * 16, 16)] = nj * t
                return 0
            lax.fori_loop(0, B, edge, 0)
            pltpu.sync_copy(xgb, agg.at[coli], add=True)
            return 0
        lax.fori_loop(0, NB, blk, 0)

        plsc.subcore_barrier()
        for z in range(SR // ZR):
            pltpu.sync_copy(agg.at[pl.ds(r0 + z * ZR, ZR)], zb)
            pltpu.sync_copy(zb, out_hbm.at[cid, pl.ds(r0 + z * ZR, ZR)])

    return k(row, col, dis, xw, ew)[:, :_N, :]


def _mm_body(x_ref, w_ref, o_ref):
    o_ref[...] = jnp.dot(x_ref[...], w_ref[...],
                         preferred_element_type=jnp.float32)


def _matmul(x, w, CB):
    n, k = x.shape
    m = w.shape[1]
    return pl.pallas_call(
        _mm_body,
        grid=(n // CB,),
        in_specs=[pl.BlockSpec((CB, k), lambda c: (c, 0)),
                  pl.BlockSpec((k, m), lambda c: (0, 0))],
        out_specs=pl.BlockSpec((CB, m), lambda c: (c, 0)),
        out_shape=jax.ShapeDtypeStruct((n, m), jnp.float32),
    )(x, w)


def _gcnD_body(hist_ref, w2t_ref, dis_ref, rdeg_ref, sfac_ref, sw2_ref):
    ones32 = jnp.ones((_NW, 1), jnp.float32)
    degT = lax.dot_general(hist_ref[...], ones32,
                           (((0,), (0,)), ((), ())),
                           preferred_element_type=jnp.float32) + 1.0
    rdeg = 1.0 / degT
    dis_ref[...] = lax.rsqrt(degT)
    rdeg_ref[...] = rdeg
    sfac_ref[...] = rdeg * rdeg
    sw2_ref[...] = jnp.sum(w2t_ref[...], axis=0, keepdims=True)


def _gcnD(hist, W2T):
    return pl.pallas_call(
        _gcnD_body,
        out_shape=[jax.ShapeDtypeStruct((_N, 1), jnp.float32),
                   jax.ShapeDtypeStruct((_N, 1), jnp.float32),
                   jax.ShapeDtypeStruct((_N, 1), jnp.float32),
                   jax.ShapeDtypeStruct((1, _HID), jnp.float32)],
    )(hist, W2T)


def _gcnB_body(agg0_ref, agg1_ref, xw_ref, rdeg_ref, sfac_ref, sw2_ref,
               bias_ref, g_ref):
    agg = agg0_ref[0] + agg1_ref[0]
    t = jnp.tanh(xw_ref[...] * sw2_ref[...])
    pre = (agg * rdeg_ref[...] + sfac_ref[...] * t + bias_ref[...])
    g_ref[...] = jax.nn.sigmoid(pre)


def _gcnB(aggp, xw, rdeg, sfac, sw2, bias, CB):
    return pl.pallas_call(
        _gcnB_body,
        grid=(_N // CB,),
        in_specs=[
            pl.BlockSpec((1, CB, _HID), lambda c: (0, c, 0)),
            pl.BlockSpec((1, CB, _HID), lambda c: (1, c, 0)),
            pl.BlockSpec((CB, _HID), lambda c: (c, 0)),
            pl.BlockSpec((CB, 1), lambda c: (c, 0)),
            pl.BlockSpec((CB, 1), lambda c: (c, 0)),
            pl.BlockSpec((1, _HID), lambda c: (0, 0)),
            pl.BlockSpec((1, _HID), lambda c: (0, 0)),
        ],
        out_specs=pl.BlockSpec((CB, _HID), lambda c: (c, 0)),
        out_shape=jax.ShapeDtypeStruct((_N, _HID), jnp.float32),
    )(aggp, aggp, xw, rdeg, sfac, sw2, bias.reshape(1, _HID))


def _gcn_sc(x, edge_index, edge_attr, W1, W2, bias):
    row = edge_index[0]
    col = edge_index[1]
    xw = _matmul(x, W1.T, CB=1000)
    ew = _matmul(edge_attr, W2.T, CB=4000)
    hist = _sc_hist(col)
    dis, rdeg, sfac, sw2 = _gcnD(hist, W2.T)
    aggp = _sc_msgs(row, col, dis.reshape(_N), xw, ew)
    return _gcnB(aggp, xw, rdeg, sfac, sw2, bias, CB=1000)


def _gcn_jax(x, edge_index, edge_attr, W1, W2, bias):
    n = x.shape[0]
    row, col = edge_index[0], edge_index[1]
    ones_e = jnp.ones(row.shape[0], dtype=x.dtype)
    deg = jax.ops.segment_sum(ones_e, col, num_segments=n) + 1.0
    dis = deg ** -0.5
    xw = x @ W1.T
    ew = edge_attr @ W2.T
    norm = dis[row] * dis[col]
    msg = norm[:, None] * jnp.tanh(xw[row] * ew)
    agg = jax.ops.segment_sum(msg, col, num_segments=n)
    sw2 = W2.sum(axis=1)
    selfmsg = (dis * dis)[:, None] * jnp.tanh(xw * sw2[None, :])
    out = (agg + selfmsg) / deg[:, None]
    return jax.nn.sigmoid(out + bias)


def kernel(x, edge_index, edge_attr, W1, W2, bias, Wih0, Whh0, Wih1, Whh1,
           Wih2, Whh2, lin_W, lin_b):
    g = _gcn_sc(x, edge_index, edge_attr, W1, W2, bias)
    W0, WA, WB1, WB2 = _splitmats(Wih0, Whh0, Wih1, Whh1, Wih2, Whh2)
    y2, hn = _lstm3s(g, W0, WA, WB1, WB2, C=400)
    return _head(y2, hn, lin_W, lin_b, C=400)


# speedup vs baseline: 10.8777x; 1.0012x over previous
import functools

import jax
import jax.numpy as jnp
import numpy as np
from jax import lax
from jax.experimental import pallas as pl
from jax.experimental.pallas import tpu as pltpu
from jax.experimental.pallas import tpu_sc as plsc

_HID = 128
_G4 = 4 * _HID
_N = 10000
_E = 320000
_NW = 32
_EPW = _E // _NW


_GSRC = {"i": 0, "f": 1, "g": 2, "o": 3}
_GDST = {"i": 0, "f": 3, "o": 6, "g": 9}


def _pack_cols(M, W, rowblk, l):
    H = _HID
    WT = W.T
    for g in ("i", "f", "g", "o"):
        src = WT[:, _GSRC[g] * H:(_GSRC[g] + 1) * H]
        c0 = (_GDST[g] + l) * H
        M = M.at[rowblk * H:(rowblk + 1) * H, c0:c0 + H].set(src)
    return M


def _bigmats(Wih0, Whh0, Wih1, Whh1, Wih2, Whh2):
    H = _HID
    WB = jnp.zeros((3 * H, 12 * H), jnp.float32)
    WB = _pack_cols(WB, Whh0, 0, 0)
    WB = _pack_cols(WB, Wih1, 0, 1)
    WB = _pack_cols(WB, Whh1, 1, 1)
    WB = _pack_cols(WB, Wih2, 1, 2)
    WB = _pack_cols(WB, Whh2, 2, 2)
    W0 = jnp.zeros((H, 12 * H), jnp.float32)
    W0 = _pack_cols(W0, Wih0, 0, 0)
    return W0, WB


def _lstm3f_body(C, n, g_ref, w0_ref, wb_ref, y_ref, hn_ref, gxscr, hcscr):
    H = _HID
    c = pl.program_id(0)
    nc = pl.num_programs(0)
    base = c * C
    gxscr[...] = jnp.dot(g_ref[...], w0_ref[...],
                         preferred_element_type=jnp.float32)

    @pl.when(c == 0)
    def _():
        hcscr[...] = jnp.zeros_like(hcscr)

    wb = wb_ref[...]
    steps = jnp.where(c == nc - 1, C + 2, C)

    def step(i, carry):
        h_all, c_all, hs0, hs1 = carry
        gx = gxscr[pl.ds(jnp.minimum(i, C - 1), 1), :]
        zz = gx + jnp.dot(h_all, wb, preferred_element_type=jnp.float32)
        sig = jax.nn.sigmoid(zz[:, 0:9 * H])
        gg = jnp.tanh(zz[:, 9 * H:12 * H])
        ia = sig[:, 0:3 * H]
        fa = sig[:, 3 * H:6 * H]
        oa = sig[:, 6 * H:9 * H]
        c_all = fa * c_all + ia * gg
        h_new = oa * jnp.tanh(c_all)
        widx = jnp.maximum(base + i - 2, 0)
        y_ref[pl.ds(widx, 1), :] = h_new[:, 2 * H:3 * H]
        gi = base + i
        hs0 = jnp.where(gi == n - 1, h_new[:, 0:H], hs0)
        hs1 = jnp.where(gi == n, h_new[:, H:2 * H], hs1)
        return (h_new, c_all, hs0, hs1)

    h_all, c_all, hs0, hs1 = jax.lax.fori_loop(
        0, steps, step,
        (hcscr[0:1, :], hcscr[1:2, :],
         hcscr[2:3, 0:H], hcscr[3:4, 0:H]))
    hcscr[0:1, :] = h_all
    hcscr[1:2, :] = c_all
    hcscr[2:3, 0:H] = hs0
    hcscr[3:4, 0:H] = hs1

    @pl.when(c == nc - 1)
    def _():
        hn_ref[0:1, :] = hs0
        hn_ref[1:2, :] = hs1
        hn_ref[2:3, :] = h_all[:, 2 * H:3 * H]


def _lstm3f(g, W0, WB, C):
    n = g.shape[0]
    nc = n // C
    body = functools.partial(_lstm3f_body, C, n)
    y2, hn = pl.pallas_call(
        body,
        grid=(nc,),
        in_specs=[
            pl.BlockSpec((C, _HID), lambda c: (c, 0)),
            pl.BlockSpec((_HID, 12 * _HID), lambda c: (0, 0)),
            pl.BlockSpec((3 * _HID, 12 * _HID), lambda c: (0, 0)),
        ],
        out_specs=[
            pl.BlockSpec((n, _HID), lambda c: (0, 0)),
            pl.BlockSpec((8, _HID), lambda c: (0, 0)),
        ],
        out_shape=[
            jax.ShapeDtypeStruct((n, _HID), jnp.float32),
            jax.ShapeDtypeStruct((8, _HID), jnp.float32),
        ],
        scratch_shapes=[
            pltpu.VMEM((C, 12 * _HID), jnp.float32),
            pltpu.VMEM((8, 3 * _HID), jnp.float32),
        ],
    )(g, W0, WB)
    return y2, hn[0:3]


def _reorder_ifog(W):
    H = _HID
    WT = W.T
    return jnp.concatenate(
        [WT[:, 0:H], WT[:, H:2 * H], WT[:, 3 * H:4 * H], WT[:, 2 * H:3 * H]],
        axis=1)


def _splitmats(Wih0, Whh0, Wih1, Whh1, Wih2, Whh2):
    W0 = _reorder_ifog(Wih0)
    WA = _reorder_ifog(Whh0).astype(jnp.bfloat16)
    WB1 = jnp.concatenate([_reorder_ifog(Wih1), _reorder_ifog(Whh1)],
                          axis=0).astype(jnp.bfloat16)
    WB2 = jnp.concatenate([_reorder_ifog(Wih2), _reorder_ifog(Whh2)],
                          axis=0).astype(jnp.bfloat16)
    return W0, WA, WB1, WB2


def _mixdot(a, b):
    return lax.dot_general(a, b, (((1,), (0,)), ((), ())),
                           preferred_element_type=jnp.float32)


def _lstm3s_body(C, n, g_ref, w0_ref, wa_ref, wb1_ref, wb2_ref,
                 y_ref, hn_ref, gxscr, hcscr):
    H = _HID
    c = pl.program_id(0)
    nc = pl.num_programs(0)
    base = c * C
    gxscr[...] = jnp.dot(g_ref[...], w0_ref[...],
                         preferred_element_type=jnp.float32)

    @pl.when(c == 0)
    def _():
        hcscr[...] = jnp.zeros_like(hcscr)

    wa = wa_ref[...]
    wb1 = wb1_ref[...]
    wb2 = wb2_ref[...]
    steps = jnp.where(c == nc - 1, C + 2, C)

    def upd(zz, cc):
        sg = jax.nn.sigmoid(zz[:, 0:3 * H])
        gg = jnp.tanh(zz[:, 3 * H:4 * H])
        ccn = sg[:, H:2 * H] * cc + sg[:, 0:H] * gg
        return sg[:, 2 * H:3 * H] * jnp.tanh(ccn), ccn

    def step(i, carry):
        h0, h1, h2, c0, c1, c2, hs0, hs1 = carry
        gx = gxscr[pl.ds(jnp.minimum(i, C - 1), 1), :]
        zz0 = gx + _mixdot(h0, wa)
        zz1 = _mixdot(jnp.concatenate([h0, h1], axis=1), wb1)
        zz2 = _mixdot(jnp.concatenate([h1, h2], axis=1), wb2)
        h0n, c0 = upd(zz0, c0)
        h1n, c1 = upd(zz1, c1)
        h2n, c2 = upd(zz2, c2)
        y_ref[pl.ds(jnp.maximum(base + i - 2, 0), 1), :] = h2n
        gi = base + i
        hs0 = jnp.where(gi == n - 1, h0n, hs0)
        hs1 = jnp.where(gi == n, h1n, hs1)
        return (h0n, h1n, h2n, c0, c1, c2, hs0, hs1)

    out = jax.lax.fori_loop(
        0, steps, step,
        tuple(hcscr[pl.ds(r, 1), :] for r in range(8)))
    for r in range(8):
        hcscr[pl.ds(r, 1), :] = out[r]

    @pl.when(c == nc - 1)
    def _():
        hn_ref[0:1, :] = out[6]
        hn_ref[1:2, :] = out[7]
        hn_ref[2:3, :] = out[2]


def _lstm3s(g, W0, WA, WB1, WB2, C):
    n = g.shape[0]
    nc = n // C
    body = functools.partial(_lstm3s_body, C, n)
    y2, hn = pl.pallas_call(
        body,
        grid=(nc,),
        in_specs=[
            pl.BlockSpec((C, _HID), lambda c: (c, 0)),
            pl.BlockSpec((_HID, _G4), lambda c: (0, 0)),
            pl.BlockSpec((_HID, _G4), lambda c: (0, 0)),
            pl.BlockSpec((2 * _HID, _G4), lambda c: (0, 0)),
            pl.BlockSpec((2 * _HID, _G4), lambda c: (0, 0)),
        ],
        out_specs=[
            pl.BlockSpec((n, _HID), lambda c: (0, 0)),
            pl.BlockSpec((8, _HID), lambda c: (0, 0)),
        ],
        out_shape=[
            jax.ShapeDtypeStruct((n, _HID), jnp.float32),
            jax.ShapeDtypeStruct((8, _HID), jnp.float32),
        ],
        scratch_shapes=[
            pltpu.VMEM((C, _G4), jnp.float32),
            pltpu.VMEM((8, _HID), jnp.float32),
        ],
    )(g, W0, WA, WB1, WB2)
    return y2, hn[0:3]


def _lstm3_body(C, g_ref, wihT_ref, whhT_ref, y_ref, hn_ref, yscr, gxscr, hcscr):
    l = pl.program_id(0)
    c = pl.program_id(1)
    base = c * C

    xin = jnp.where(l == 0, g_ref[...], yscr[pl.ds(base, C), :])
    gxscr[...] = jnp.dot(xin, wihT_ref[0], preferred_element_type=jnp.float32)

    @pl.when(c == 0)
    def _():
        hcscr[...] = jnp.zeros_like(hcscr)

    whh = whhT_ref[0]

    def step(t, carry):
        h, cc = carry
        gates = gxscr[pl.ds(t, 1), :] + jnp.dot(h, whh,
                                                preferred_element_type=jnp.float32)
        i = jax.nn.sigmoid(gates[:, 0:_HID])
        f = jax.nn.sigmoid(gates[:, _HID:2 * _HID])
        gg = jnp.tanh(gates[:, 2 * _HID:3 * _HID])
        o = jax.nn.sigmoid(gates[:, 3 * _HID:4 * _HID])
        cc = f * cc + i * gg
        h = o * jnp.tanh(cc)
        yscr[pl.ds(base + t, 1), :] = h
        return (h, cc)

    h, cc = jax.lax.fori_loop(0, C, step, (hcscr[0:1, :], hcscr[1:2, :]))
    hcscr[0:1, :] = h
    hcscr[1:2, :] = cc
    y_ref[...] = yscr[pl.ds(base, C), :]

    @pl.when(c == pl.num_programs(1) - 1)
    def _():
        hn_ref[pl.ds(l, 1), :] = h


def _lstm3(g, wihT, whhT, C):
    n = g.shape[0]
    nc = n // C
    body = functools.partial(_lstm3_body, C)
    y2, hn = pl.pallas_call(
        body,
        grid=(3, nc),
        in_specs=[
            pl.BlockSpec((C, _HID), lambda l, c: (c, 0)),
            pl.BlockSpec((1, _HID, _G4), lambda l, c: (l, 0, 0)),
            pl.BlockSpec((1, _HID, _G4), lambda l, c: (l, 0, 0)),
        ],
        out_specs=[
            pl.BlockSpec((C, _HID), lambda l, c: (c, 0)),
            pl.BlockSpec((8, _HID), lambda l, c: (0, 0)),
        ],
        out_shape=[
            jax.ShapeDtypeStruct((n, _HID), jnp.float32),
            jax.ShapeDtypeStruct((8, _HID), jnp.float32),
        ],
        scratch_shapes=[
            pltpu.VMEM((n, _HID), jnp.float32),
            pltpu.VMEM((C, _G4), jnp.float32),
            pltpu.VMEM((8, _HID), jnp.float32),
        ],
    )(g, wihT, whhT)
    return y2, hn[0:3]


def _head_body(C, y2_ref, hn_ref, linmp_ref, linap_ref, linhn_ref, linb_ref,
               out_ref, accs):
    c = pl.program_id(0)

    @pl.when(c == 0)
    def _():
        accs[...] = jnp.zeros_like(accs)

    y2 = y2_ref[...]
    rolled = pltpu.roll(y2, _HID - 1, 1)
    sel = jax.lax.broadcasted_iota(jnp.int32, (C, _HID), 1) % 2 == 0
    smat = (jax.lax.broadcasted_iota(jnp.int32, (_HID, _HID // 2), 0) ==
            2 * jax.lax.broadcasted_iota(jnp.int32, (_HID, _HID // 2), 1)
            ).astype(jnp.float32)
    pairmax = jnp.maximum(y2, rolled)
    pairsum = y2 + rolled
    mp = jnp.dot(jnp.where(sel, pairmax, 0.0), smat,
                 preferred_element_type=jnp.float32)
    ap = 0.5 * jnp.dot(jnp.where(sel, pairsum, 0.0), smat,
                       preferred_element_type=jnp.float32)

    accs[0:3, 0:64] += jnp.sum(linmp_ref[...] * mp[None, :, :], axis=1)
    accs[0:3, 64:128] += jnp.sum(linap_ref[...] * ap[None, :, :], axis=1)

    @pl.when(c == pl.num_programs(0) - 1)
    def _():
        hn = hn_ref[...]
        prod = linhn_ref[...] * hn[None, :, :]
        s2 = jnp.sum(prod, axis=2)
        hnpart = jnp.sum(s2, axis=1, keepdims=True)
        logits = (jnp.sum(accs[0:3, :], axis=1, keepdims=True)
                  + hnpart + linb_ref[...])
        m = jnp.max(logits, axis=0, keepdims=True)
        e = jnp.exp(logits - m)
        out_ref[0:3, 0:1] = e / jnp.sum(e, axis=0, keepdims=True)


def _head(y2, hn, lin_W, lin_b, C):
    n = y2.shape[0]
    nc = n // C
    P = _HID // 2
    linmp = lin_W[:, :n * P].reshape(3, n, P)
    linap = lin_W[:, n * P:2 * n * P].reshape(3, n, P)
    linhn = lin_W[:, 2 * n * P:].reshape(3, 3, _HID)
    body = functools.partial(_head_body, C)
    out = pl.pallas_call(
        body,
        grid=(nc,),
        in_specs=[
            pl.BlockSpec((C, _HID), lambda c: (c, 0)),
            pl.BlockSpec((3, _HID), lambda c: (0, 0)),
            pl.BlockSpec((3, C, P), lambda c: (0, c, 0)),
            pl.BlockSpec((3, C, P), lambda c: (0, c, 0)),
            pl.BlockSpec((3, 3, _HID), lambda c: (0, 0, 0)),
            pl.BlockSpec((3, 1), lambda c: (0, 0)),
        ],
        out_specs=pl.BlockSpec((8, 128), lambda c: (0, 0)),
        out_shape=jax.ShapeDtypeStruct((8, 128), jnp.float32),
        scratch_shapes=[pltpu.VMEM((8, 128), jnp.float32)],
    )(y2, hn, linmp, linap, linhn, lin_b.reshape(3, 1))
    return out[0:3, 0]


def _sc_hist(col):
    mesh = plsc.VectorSubcoreMesh(core_axis_name="c", subcore_axis_name="s")
    CH = 2000

    @functools.partial(
        pl.kernel, mesh=mesh,
        compiler_params=pltpu.CompilerParams(needs_layout_passes=False),
        out_type=jax.ShapeDtypeStruct((_NW, _N), jnp.float32),
        scratch_types=[
            pltpu.VMEM((CH,), jnp.int32),
            pltpu.VMEM((_N,), jnp.float32),
        ],
    )
    def k(col_hbm, out_hbm, colv, hist):
        cid = lax.axis_index("c")
        sid = lax.axis_index("s")
        wid = cid * 16 + sid
        base = wid * _EPW
        zeros = jnp.zeros((16,), jnp.float32)
        ones = jnp.ones((16,), jnp.float32)

        def zloop(i, _):
            hist[pl.ds(i * 16, 16)] = zeros
            return 0
        lax.fori_loop(0, _N // 16, zloop, 0)

        def chunk(cb, _):
            pltpu.sync_copy(col_hbm.at[pl.ds(base + cb * CH, CH)], colv)

            def inner(j, _):
                idx = colv[pl.ds(j * 16, 16)]
                plsc.addupdate_scatter(hist, [idx], ones)
                return 0
            lax.fori_loop(0, CH // 16, inner, 0)
            return 0
        lax.fori_loop(0, _EPW // CH, chunk, 0)
        pltpu.sync_copy(hist, out_hbm.at[wid])

    return k(col)


def _sc_msgs(row, col, dis, xw, ew):
    mesh = plsc.VectorSubcoreMesh(core_axis_name="c", subcore_axis_name="s")
    B = 80
    NB = _EPW // B
    NP = 10240
    SR = NP // 16
    ZR = 128

    @functools.partial(
        pl.kernel, mesh=mesh,
        compiler_params=pltpu.CompilerParams(needs_layout_passes=False),
        out_type=jax.ShapeDtypeStruct((2, NP, _HID), jnp.float32),
        scratch_types=[
            pltpu.VMEM((_N,), jnp.float32),
            pltpu.VMEM((B,), jnp.int32),
            pltpu.VMEM((B,), jnp.int32),
            pltpu.VMEM((B + 16,), jnp.float32),
            pltpu.VMEM((B, _HID), jnp.float32),
            pltpu.VMEM((B, _HID), jnp.float32),
            pltpu.VMEM((ZR, _HID), jnp.float32),
            pltpu.VMEM_SHARED((NP, _HID), jnp.float32),
            pltpu.SemaphoreType.DMA,
        ],
    )
    def k(row_hbm, col_hbm, dis_hbm, xw_hbm, ew_hbm, out_hbm,
          disv, rowi, coli, normv, ewb, xgb, zb, agg, sem):
        cid = lax.axis_index("c")
        sid = lax.axis_index("s")
        wid = cid * 16 + sid
        base = wid * _EPW
        pltpu.sync_copy(dis_hbm, disv)

        zeros = jnp.zeros((16,), jnp.float32)

        def zrow(i, _):
            for kk in range(_HID // 16):
                zb[i, pl.ds(kk * 16, 16)] = zeros
            return 0
        lax.fori_loop(0, ZR, zrow, 0)
        r0 = sid * SR
        for z in range(SR // ZR):
            pltpu.sync_copy(zb, agg.at[pl.ds(r0 + z * ZR, ZR)])
        plsc.subcore_barrier()

        def blk(b, _):
            e0 = base + b * B
            pltpu.sync_copy(row_hbm.at[pl.ds(e0, B)], rowi)
            pltpu.sync_copy(col_hbm.at[pl.ds(e0, B)], coli)
            pltpu.sync_copy(ew_hbm.at[pl.ds(e0, B)], ewb)
            pltpu.async_copy(xw_hbm.at[rowi], xgb, sem).wait()

            def nrm(j, _):
                ri = rowi[pl.ds(j * 16, 16)]
                ci = coli[pl.ds(j * 16, 16)]
                dr = plsc.load_gather(disv, [ri])
                dc = plsc.load_gather(disv, [ci])
                normv[pl.ds(j * 16, 16)] = dr * dc
                return 0
            lax.fori_loop(0, B // 16, nrm, 0)

            def edge(j, _):
                nj = normv[pl.ds(j, 16)][0]
                for kk in range(_HID // 16):
                    xv = xgb[j, pl.ds(kk * 16, 16)]
                    ev = ewb[j, pl.ds(kk * 16, 16)]
                    p = xv * ev
                    e2 = jnp.exp(p + p)
                    t = 1.0 - 2.0 / (e2 + 1.0)
                    xgb[j, pl.ds(kk * 16, 16)] = nj * t
                return 0
            lax.fori_loop(0, B, edge, 0)
            pltpu.sync_copy(xgb, agg.at[coli], add=True)
            return 0
        lax.fori_loop(0, NB, blk, 0)

        plsc.subcore_barrier()
        for z in range(SR // ZR):
            pltpu.sync_copy(agg.at[pl.ds(r0 + z * ZR, ZR)], zb)
            pltpu.sync_copy(zb, out_hbm.at[cid, pl.ds(r0 + z * ZR, ZR)])

    return k(row, col, dis, xw, ew)[:, :_N, :]


def _mm_body(x_ref, w_ref, o_ref):
    o_ref[...] = jnp.dot(x_ref[...], w_ref[...],
                         preferred_element_type=jnp.float32)


def _matmul(x, w, CB):
    n, k = x.shape
    m = w.shape[1]
    return pl.pallas_call(
        _mm_body,
        grid=(n // CB,),
        in_specs=[pl.BlockSpec((CB, k), lambda c: (c, 0)),
                  pl.BlockSpec((k, m), lambda c: (0, 0))],
        out_specs=pl.BlockSpec((CB, m), lambda c: (c, 0)),
        out_shape=jax.ShapeDtypeStruct((n, m), jnp.float32),
    )(x, w)


def _gcnD_body(hist_ref, w2t_ref, dis_ref, rdeg_ref, sfac_ref, sw2_ref):
    ones32 = jnp.ones((_NW, 1), jnp.float32)
    degT = lax.dot_general(hist_ref[...], ones32,
                           (((0,), (0,)), ((), ())),
                           preferred_element_type=jnp.float32) + 1.0
    rdeg = 1.0 / degT
    dis_ref[...] = lax.rsqrt(degT)
    rdeg_ref[...] = rdeg
    sfac_ref[...] = rdeg * rdeg
    sw2_ref[...] = jnp.sum(w2t_ref[...], axis=0, keepdims=True)


def _gcnD(hist, W2T):
    return pl.pallas_call(
        _gcnD_body,
        out_shape=[jax.ShapeDtypeStruct((_N, 1), jnp.float32),
                   jax.ShapeDtypeStruct((_N, 1), jnp.float32),
                   jax.ShapeDtypeStruct((_N, 1), jnp.float32),
                   jax.ShapeDtypeStruct((1, _HID), jnp.float32)],
    )(hist, W2T)


def _gcnB_body(agg0_ref, agg1_ref, xw_ref, rdeg_ref, sfac_ref, sw2_ref,
               bias_ref, g_ref):
    agg = agg0_ref[0] + agg1_ref[0]
    t = jnp.tanh(xw_ref[...] * sw2_ref[...])
    pre = (agg * rdeg_ref[...] + sfac_ref[...] * t + bias_ref[...])
    g_ref[...] = jax.nn.sigmoid(pre)


def _gcnB(aggp, xw, rdeg, sfac, sw2, bias, CB):
    return pl.pallas_call(
        _gcnB_body,
        grid=(_N // CB,),
        in_specs=[
            pl.BlockSpec((1, CB, _HID), lambda c: (0, c, 0)),
            pl.BlockSpec((1, CB, _HID), lambda c: (1, c, 0)),
            pl.BlockSpec((CB, _HID), lambda c: (c, 0)),
            pl.BlockSpec((CB, 1), lambda c: (c, 0)),
            pl.BlockSpec((CB, 1), lambda c: (c, 0)),
            pl.BlockSpec((1, _HID), lambda c: (0, 0)),
            pl.BlockSpec((1, _HID), lambda c: (0, 0)),
        ],
        out_specs=pl.BlockSpec((CB, _HID), lambda c: (c, 0)),
        out_shape=jax.ShapeDtypeStruct((_N, _HID), jnp.float32),
    )(aggp, aggp, xw, rdeg, sfac, sw2, bias.reshape(1, _HID))


def _gcn_sc(x, edge_index, edge_attr, W1, W2, bias):
    row = edge_index[0]
    col = edge_index[1]
    xw = _matmul(x, W1.T, CB=1000)
    ew = _matmul(edge_attr, W2.T, CB=4000)
    hist = _sc_hist(col)
    dis, rdeg, sfac, sw2 = _gcnD(hist, W2.T)
    aggp = _sc_msgs(row, col, dis.reshape(_N), xw, ew)
    return _gcnB(aggp, xw, rdeg, sfac, sw2, bias, CB=1000)


def _gcn_jax(x, edge_index, edge_attr, W1, W2, bias):
    n = x.shape[0]
    row, col = edge_index[0], edge_index[1]
    ones_e = jnp.ones(row.shape[0], dtype=x.dtype)
    deg = jax.ops.segment_sum(ones_e, col, num_segments=n) + 1.0
    dis = deg ** -0.5
    xw = x @ W1.T
    ew = edge_attr @ W2.T
    norm = dis[row] * dis[col]
    msg = norm[:, None] * jnp.tanh(xw[row] * ew)
    agg = jax.ops.segment_sum(msg, col, num_segments=n)
    sw2 = W2.sum(axis=1)
    selfmsg = (dis * dis)[:, None] * jnp.tanh(xw * sw2[None, :])
    out = (agg + selfmsg) / deg[:, None]
    return jax.nn.sigmoid(out + bias)


def kernel(x, edge_index, edge_attr, W1, W2, bias, Wih0, Whh0, Wih1, Whh1,
           Wih2, Whh2, lin_W, lin_b):
    g = _gcn_sc(x, edge_index, edge_attr, W1, W2, bias)
    W0, WA, WB1, WB2 = _splitmats(Wih0, Whh0, Wih1, Whh1, Wih2, Whh2)
    y2, hn = _lstm3s(g, W0, WA, WB1, WB2, C=1000)
    return _head(y2, hn, lin_W, lin_b, C=400)
